# baseline (speedup 1.0000x reference)
import functools

import jax
import jax.numpy as jnp
from jax import lax
from jax.experimental import pallas as pl
from jax.experimental.pallas import tpu as pltpu
from jax.experimental.pallas import tpu_sc as plsc

_info = plsc.get_sparse_core_info()
_NC, _NS, _L = _info.num_cores, _info.num_subcores, _info.num_lanes
_NW = _NC * _NS

_B, _N = 4096, 512
_NB = _N // 128
_ROWS_W = _B // _NW
_B_ROWS = 32
_N_BLK = _ROWS_W // _B_ROWS
_GROUPS = _B_ROWS * _N // 16
_IN_BUFS = 2
_OUT_BUFS = 2


def _sc_lookup(diag4, grid_flat):
    mesh = plsc.VectorSubcoreMesh(core_axis_name="c", subcore_axis_name="s")

    @functools.partial(
        pl.kernel,
        mesh=mesh,
        out_type=jax.ShapeDtypeStruct((_B // 8, _NB, 8, 128), jnp.float32),
        compiler_params=pltpu.CompilerParams(
            needs_layout_passes=False, use_tc_tiling_on_sc=False
        ),
        scratch_types=[
            pltpu.VMEM((256,), jnp.float32),
            *[pltpu.VMEM((_B_ROWS, _NB, 2, 128), jnp.float32)] * _IN_BUFS,
            *[pltpu.VMEM((_B_ROWS // 8, _NB, 8, 128), jnp.float32)]
            * _OUT_BUFS,
            *[pltpu.SemaphoreType.DMA] * (_IN_BUFS + _OUT_BUFS + 1),
        ],
    )
    def k(diag_hbm, grid_hbm, out_hbm, table_v, *bufs):
        in_bufs = bufs[:_IN_BUFS]
        out_bufs = bufs[_IN_BUFS:_IN_BUFS + _OUT_BUFS]
        isems = bufs[_IN_BUFS + _OUT_BUFS:_IN_BUFS + _OUT_BUFS + _IN_BUFS]
        osems = bufs[2 * _IN_BUFS + _OUT_BUFS:]
        tsem = osems[_OUT_BUFS]
        osems = osems[:_OUT_BUFS]

        wid = lax.axis_index("s") * _NC + lax.axis_index("c")
        base_row = wid * _ROWS_W

        def in_row0(b):
            return base_row + b * _B_ROWS

        table_dma = pltpu.async_copy(grid_hbm, table_v, tsem)
        in_dma = [None] * _N_BLK
        for b in range(_IN_BUFS - 1):
            in_dma[b] = pltpu.async_copy(
                diag_hbm.at[pl.ds(in_row0(b), _B_ROWS)],
                in_bufs[b % _IN_BUFS],
                isems[b % _IN_BUFS],
            )
        table_dma.wait()

        out_dma = [None] * _N_BLK
        for b in range(_N_BLK):
            in_v = in_bufs[b % _IN_BUFS]
            out_v = out_bufs[b % _OUT_BUFS]
            nxt = b + _IN_BUFS - 1
            if nxt < _N_BLK:
                in_dma[nxt] = pltpu.async_copy(
                    diag_hbm.at[pl.ds(in_row0(nxt), _B_ROWS)],
                    in_bufs[nxt % _IN_BUFS],
                    isems[nxt % _IN_BUFS],
                )
            in_dma[b].wait()
            if b >= _OUT_BUFS:
                out_dma[b - _OUT_BUFS].wait()

            @plsc.parallel_loop(0, _GROUPS, unroll=4)
            def grp(g):
                r = g >> 5
                rem = g & 31
                cb = rem >> 3
                c16 = (rem & 7) * 16
                xs = in_v[r, cb, 0, pl.ds(c16, 16)]
                ys = in_v[r, cb, 1, pl.ds(c16, 16)]
                ix = (xs * 16.0).astype(jnp.int32)
                iy = (ys * 16.0).astype(jnp.int32)
                w = plsc.load_gather(table_v, [(ix << 4) | iy])
                out_v[r >> 3, cb, r & 7, pl.ds(c16, 16)] = w

            out_dma[b] = pltpu.async_copy(
                out_v,
                out_hbm.at[pl.ds(in_row0(b) // 8, _B_ROWS // 8)],
                osems[b % _OUT_BUFS],
            )
        for b in range(max(_N_BLK - _OUT_BUFS, 0), _N_BLK):
            out_dma[b].wait()

    return k(diag4, grid_flat)


def kernel(diagrams, grid):
    diag4 = diagrams.reshape(_B, _NB, 128, 2).transpose(0, 1, 3, 2)
    out4 = _sc_lookup(diag4, grid.reshape(-1))
    return out4.transpose(0, 2, 1, 3).reshape(_B, _N)


# speedup vs baseline: 1.0620x; 1.0620x over previous
import functools

import jax
import jax.numpy as jnp
from jax import lax
from jax.experimental import pallas as pl
from jax.experimental.pallas import tpu as pltpu
from jax.experimental.pallas import tpu_sc as plsc

_info = plsc.get_sparse_core_info()
_NC, _NS, _L = _info.num_cores, _info.num_subcores, _info.num_lanes
_NW = _NC * _NS

_B, _N = 4096, 512
_NB = _N // 128
_ROWS_W = _B // _NW
_B_ROWS = 32
_N_BLK = _ROWS_W // _B_ROWS
_GROUPS = _B_ROWS * _N // 16
_IN_BUFS = 2
_OUT_BUFS = 2


def _sc_lookup(diag4, grid_flat):
    mesh = plsc.VectorSubcoreMesh(core_axis_name="c", subcore_axis_name="s")

    @functools.partial(
        pl.kernel,
        mesh=mesh,
        out_type=jax.ShapeDtypeStruct((_B // 8, _NB, 8, 128), jnp.float32),
        compiler_params=pltpu.CompilerParams(
            needs_layout_passes=False, use_tc_tiling_on_sc=False
        ),
        scratch_types=[
            pltpu.VMEM((256,), jnp.float32),
            *[pltpu.VMEM((_B_ROWS, _NB, 2, 128), jnp.float32)] * _IN_BUFS,
            *[pltpu.VMEM((_B_ROWS // 8, _NB, 8, 128), jnp.float32)]
            * _OUT_BUFS,
            *[pltpu.SemaphoreType.DMA] * (_IN_BUFS + _OUT_BUFS + 1),
        ],
    )
    def k(diag_hbm, grid_hbm, out_hbm, table_v, *bufs):
        in_bufs = bufs[:_IN_BUFS]
        out_bufs = bufs[_IN_BUFS:_IN_BUFS + _OUT_BUFS]
        isems = bufs[_IN_BUFS + _OUT_BUFS:_IN_BUFS + _OUT_BUFS + _IN_BUFS]
        osems = bufs[2 * _IN_BUFS + _OUT_BUFS:]
        tsem = osems[_OUT_BUFS]
        osems = osems[:_OUT_BUFS]

        wid = lax.axis_index("s") * _NC + lax.axis_index("c")
        base_row = wid * _ROWS_W

        def in_row0(b):
            return base_row + b * _B_ROWS

        table_dma = pltpu.async_copy(grid_hbm, table_v, tsem)
        in_dma = [None] * _N_BLK
        for b in range(_IN_BUFS - 1):
            in_dma[b] = pltpu.async_copy(
                diag_hbm.at[pl.ds(in_row0(b), _B_ROWS)],
                in_bufs[b % _IN_BUFS],
                isems[b % _IN_BUFS],
            )
        table_dma.wait()

        out_dma = [None] * _N_BLK
        for b in range(_N_BLK):
            in_v = in_bufs[b % _IN_BUFS]
            out_v = out_bufs[b % _OUT_BUFS]
            nxt = b + _IN_BUFS - 1
            if nxt < _N_BLK:
                in_dma[nxt] = pltpu.async_copy(
                    diag_hbm.at[pl.ds(in_row0(nxt), _B_ROWS)],
                    in_bufs[nxt % _IN_BUFS],
                    isems[nxt % _IN_BUFS],
                )
            in_dma[b].wait()
            if b >= _OUT_BUFS:
                out_dma[b - _OUT_BUFS].wait()

            @plsc.parallel_loop(0, _GROUPS, unroll=8)
            def grp(g):
                r = g >> 5
                rem = g & 31
                cb = rem >> 3
                c16 = (rem & 7) * 16
                xs = in_v[r, cb, 0, pl.ds(c16, 16)]
                ys = in_v[r, cb, 1, pl.ds(c16, 16)]
                ix = (xs * 16.0).astype(jnp.int32)
                iy = (ys * 16.0).astype(jnp.int32)
                w = plsc.load_gather(table_v, [(ix << 4) | iy])
                out_v[r >> 3, cb, r & 7, pl.ds(c16, 16)] = w

            out_dma[b] = pltpu.async_copy(
                out_v,
                out_hbm.at[pl.ds(in_row0(b) // 8, _B_ROWS // 8)],
                osems[b % _OUT_BUFS],
            )
        for b in range(max(_N_BLK - _OUT_BUFS, 0), _N_BLK):
            out_dma[b].wait()

    return k(diag4, grid_flat)


def kernel(diagrams, grid):
    diag4 = diagrams.reshape(_B, _NB, 128, 2).transpose(0, 1, 3, 2)
    out4 = _sc_lookup(diag4, grid.reshape(-1))
    return out4.transpose(0, 2, 1, 3).reshape(_B, _N)
